# baseline (speedup 1.0000x reference)
import functools

import jax
import jax.numpy as jnp
from jax import lax
from jax.experimental import pallas as pl
from jax.experimental.pallas import tpu as pltpu
from jax.experimental.pallas import tpu_sc as plsc

V = 1000000
D = 64
OUT = 2
B = 4096
L = 200

_NC = 2
_NS = 16
_NW = _NC * _NS
_BPW = B // _NW
_C0 = 128
_C1 = L - _C0
_NBUF = 4


def _sc_pool_body(text_hbm, table_hbm, out_hbm, idx_v, bufs, pooled_v, sems):
    wid = lax.axis_index("s") * _NC + lax.axis_index("c")
    base = wid * _BPW

    pltpu.sync_copy(text_hbm.at[pl.ds(base, _BPW)], idx_v)

    def issue(r, b):
        pltpu.async_copy(
            table_hbm.at[idx_v.at[r, pl.ds(0, _C0)]],
            bufs[b].at[pl.ds(0, _C0)], sems[b])
        pltpu.async_copy(
            table_hbm.at[idx_v.at[r, pl.ds(_C0, _C1)]],
            bufs[b].at[pl.ds(_C0, _C1)], sems[b])

    def drain(b):
        pltpu.make_async_copy(
            table_hbm.at[idx_v.at[0, pl.ds(0, _C0)]],
            bufs[b].at[pl.ds(0, _C0)], sems[b]).wait()
        pltpu.make_async_copy(
            table_hbm.at[idx_v.at[0, pl.ds(0, _C1)]],
            bufs[b].at[pl.ds(_C0, _C1)], sems[b]).wait()

    def acc_row(b, r):
        buf = bufs[b]

        def tok8(t8, acc):
            accs = list(acc)
            for u in range(8):
                t = t8 * 8 + u
                for j in range(4):
                    accs[j] = accs[j] + buf[t, pl.ds(16 * j, 16)]
            return tuple(accs)

        z = jnp.zeros((16,), jnp.float32)
        accs = lax.fori_loop(0, L // 8, tok8, (z, z, z, z))
        for j in range(4):
            pooled_v[r, pl.ds(16 * j, 16)] = accs[j]

    for b in range(_NBUF):
        issue(b, b)

    def grp_loop(i, carry):
        del carry
        for b in range(_NBUF):
            r = _NBUF * i + b
            drain(b)

            @pl.when(r + _NBUF < _BPW)
            def _():
                issue(r + _NBUF, b)

            acc_row(b, r)
        return 0

    lax.fori_loop(0, _BPW // _NBUF, grp_loop, 0)
    pltpu.sync_copy(pooled_v, out_hbm.at[pl.ds(base, _BPW)])


@functools.partial(
    pl.kernel,
    mesh=plsc.VectorSubcoreMesh(core_axis_name="c", subcore_axis_name="s"),
    out_type=jax.ShapeDtypeStruct((B, D), jnp.float32),
    scratch_types=[
        pltpu.VMEM((_BPW, 256), jnp.int32),
        [pltpu.VMEM((L, D), jnp.float32) for _ in range(_NBUF)],
        pltpu.VMEM((_BPW, D), jnp.float32),
        [pltpu.SemaphoreType.DMA for _ in range(_NBUF)],
    ],
    compiler_params=pltpu.CompilerParams(use_tc_tiling_on_sc=False),
)
def _sc_pool(text_hbm, table_hbm, out_hbm, idx_v, bufs, pooled_v, sems):
    _sc_pool_body(text_hbm, table_hbm, out_hbm, idx_v, bufs, pooled_v, sems)


def _fc_body(pooled_ref, w_ref, b_ref, out_ref):
    pooled = pooled_ref[...] * jnp.float32(1.0 / L)
    out_ref[...] = (
        jnp.dot(pooled, w_ref[...].T, preferred_element_type=jnp.float32)
        + b_ref[...]
    )


def _fc(pooled_sums, fc_W, fc_b):
    return pl.pallas_call(
        _fc_body,
        out_shape=jax.ShapeDtypeStruct((B, OUT), jnp.float32),
    )(pooled_sums, fc_W, fc_b.reshape(1, OUT))


def kernel(text, W_emb, fc_W, fc_b):
    t32 = text.astype(jnp.int32)
    text256 = jnp.concatenate([t32, t32[:, :56]], axis=1)
    pooled_sums = _sc_pool(text256, W_emb)
    return _fc(pooled_sums, fc_W, fc_b)


# speedup vs baseline: 1.0025x; 1.0025x over previous
import functools

import jax
import jax.numpy as jnp
from jax import lax
from jax.experimental import pallas as pl
from jax.experimental.pallas import tpu as pltpu
from jax.experimental.pallas import tpu_sc as plsc

V = 1000000
D = 64
OUT = 2
B = 4096
L = 200

_NC = 2
_NS = 16
_NW = _NC * _NS
_BPW = B // _NW
_C0 = 128
_C1 = L - _C0
_NBUF = 4


def _sc_pool_body(idxa_hbm, idxb_hbm, table_hbm, out_hbm,
                  idxa_v, idxb_v, bufs, pooled_v, sems):
    wid = lax.axis_index("s") * _NC + lax.axis_index("c")
    base = wid * _BPW

    pltpu.sync_copy(idxa_hbm.at[pl.ds(base * _C0, _BPW * _C0)], idxa_v)
    pltpu.sync_copy(idxb_hbm.at[pl.ds(base * _C0, _BPW * _C0)], idxb_v)

    def issue(r, b):
        pltpu.async_copy(
            table_hbm.at[idxa_v.at[pl.ds(r * _C0, _C0)]],
            bufs[b].at[pl.ds(0, _C0)], sems[b])
        pltpu.async_copy(
            table_hbm.at[idxb_v.at[pl.ds(r * _C0, _C1)]],
            bufs[b].at[pl.ds(_C0, _C1)], sems[b])

    def drain(b):
        pltpu.make_async_copy(
            table_hbm.at[idxa_v.at[pl.ds(0, _C0)]],
            bufs[b].at[pl.ds(0, _C0)], sems[b]).wait()
        pltpu.make_async_copy(
            table_hbm.at[idxb_v.at[pl.ds(0, _C1)]],
            bufs[b].at[pl.ds(_C0, _C1)], sems[b]).wait()

    def acc_row(b, r):
        buf = bufs[b]

        def tok8(t8, acc):
            accs = list(acc)
            for u in range(8):
                t = t8 * 8 + u
                for j in range(4):
                    accs[j] = accs[j] + buf[t, pl.ds(16 * j, 16)]
            return tuple(accs)

        z = jnp.zeros((16,), jnp.float32)
        accs = lax.fori_loop(0, L // 8, tok8, (z, z, z, z))
        for j in range(4):
            pooled_v[r, pl.ds(16 * j, 16)] = accs[j]

    for b in range(_NBUF):
        issue(b, b)

    def grp_loop(i, carry):
        del carry
        for b in range(_NBUF):
            r = _NBUF * i + b
            drain(b)

            @pl.when(r + _NBUF < _BPW)
            def _():
                issue(r + _NBUF, b)

            acc_row(b, r)
        return 0

    lax.fori_loop(0, _BPW // _NBUF, grp_loop, 0)
    pltpu.sync_copy(pooled_v, out_hbm.at[pl.ds(base, _BPW)])


@functools.partial(
    pl.kernel,
    mesh=plsc.VectorSubcoreMesh(core_axis_name="c", subcore_axis_name="s"),
    out_type=jax.ShapeDtypeStruct((B, D), jnp.float32),
    scratch_types=[
        pltpu.VMEM((_BPW * _C0,), jnp.int32),
        pltpu.VMEM((_BPW * _C0,), jnp.int32),
        [pltpu.VMEM((L, D), jnp.float32) for _ in range(_NBUF)],
        pltpu.VMEM((_BPW, D), jnp.float32),
        [pltpu.SemaphoreType.DMA for _ in range(_NBUF)],
    ],
    compiler_params=pltpu.CompilerParams(use_tc_tiling_on_sc=False),
)
def _sc_pool(idxa_hbm, idxb_hbm, table_hbm, out_hbm,
             idxa_v, idxb_v, bufs, pooled_v, sems):
    _sc_pool_body(idxa_hbm, idxb_hbm, table_hbm, out_hbm,
                  idxa_v, idxb_v, bufs, pooled_v, sems)


def _fc_body(pooled_ref, w_ref, b_ref, out_ref):
    pooled = pooled_ref[...] * jnp.float32(1.0 / L)
    out_ref[...] = (
        jnp.dot(pooled, w_ref[...].T, preferred_element_type=jnp.float32)
        + b_ref[...]
    )


def _fc(pooled_sums, fc_W, fc_b):
    return pl.pallas_call(
        _fc_body,
        out_shape=jax.ShapeDtypeStruct((B, OUT), jnp.float32),
    )(pooled_sums, fc_W, fc_b.reshape(1, OUT))


def kernel(text, W_emb, fc_W, fc_b):
    t32 = text.astype(jnp.int32)
    flat_a = t32[:, :_C0].reshape(-1)
    flat_b = jnp.concatenate([t32[:, _C0:], t32[:, :_C0 - _C1]], axis=1).reshape(-1)
    pooled_sums = _sc_pool(flat_a, flat_b, W_emb)
    return _fc(pooled_sums, fc_W, fc_b)
